# baseline (speedup 1.0000x reference)
import jax
import jax.numpy as jnp
from jax import lax
from jax.experimental import pallas as pl
from jax.experimental.pallas import tpu as pltpu

_BATCH = 4096
_FEAT = 8192
_RB = 256
_BB = 64
_D2 = 1024

_KS0 = 0
_KS1 = 42
_KS2 = (0x1BD11BDA ^ _KS0 ^ _KS1) & 0xFFFFFFFF
_ROT_A = (13, 15, 26, 6)
_ROT_B = (17, 29, 16, 24)


def _c(v):
    return jnp.int32(v if v < 2**31 else v - 2**32)


def _rotl(x, r):
    return lax.shift_left(x, _c(r)) | lax.shift_right_logical(x, _c(32 - r))


def _mix(x0, x1, rots):
    for r in rots:
        x0 = x0 + x1
        x1 = _rotl(x1, r)
        x1 = x1 ^ x0
    return x0, x1


def _keep_mask(x1):
    x0 = x1
    x1 = _rotl(x1, _ROT_A[0]) ^ x0
    x0, x1 = _mix(x0, x1, _ROT_A[1:])
    x0, x1 = x0 + _c(_KS1), x1 + _c(_KS2 + 1)
    x0, x1 = _mix(x0, x1, _ROT_B)
    x0, x1 = x0 + _c(_KS2), x1 + _c(_KS0 + 2)
    x0, x1 = _mix(x0, x1, _ROT_A)
    x0, x1 = x0 + _c(_KS0), x1 + _c(_KS1 + 3)
    x0, x1 = _mix(x0, x1, _ROT_B)
    x0, x1 = x0 + _c(_KS1), x1 + _c(_KS2 + 4)
    x0, x1 = _mix(x0, x1, _ROT_A)
    x0, x1 = x0 + _c(_KS2), x1 + _c(_KS0 + 5)
    bits = x0 ^ x1
    return bits >= 0


def _rank_body(s_row_ref, s_col_ref, rank_ref):
    i = pl.program_id(0)
    s_all = s_row_ref[...]
    s_col = s_col_ref[...]
    gt = (s_all > s_col).astype(jnp.int32)
    idx_all = lax.broadcasted_iota(jnp.int32, (_RB, _BATCH), 1)
    row_ids = i * _RB + lax.broadcasted_iota(jnp.int32, (_RB, 1), 0)
    tie = ((s_all == s_col) & (idx_all < row_ids)).astype(jnp.int32)
    rank_ref[...] = jnp.sum(gt + tie, axis=1, keepdims=True)


def _apply_body(rank_ref, x_ref, o_ref):
    r = rank_ref[...]
    d1 = lax.broadcasted_iota(jnp.int32, (1, 8, _D2), 1)
    d2 = lax.broadcasted_iota(jnp.int32, (1, 8, _D2), 2)
    offs = (lax.shift_left((d2 >> 7) & 1, _c(12))
            + lax.shift_left(d1 >> 2, _c(11))
            + lax.shift_left(d2 & 127, _c(4))
            + lax.shift_left(d2 >> 8, _c(2))
            + (d1 & 3))
    base = (lax.shift_left(r, _c(13)) + _c(_KS1)).reshape(_BB, 1, 1)
    x1 = base + offs
    keep = _keep_mask(x1)
    x = x_ref[...]
    o_ref[...] = jnp.where(keep, x + x, jnp.zeros_like(x))


def kernel(input, W, b):
    gap = jnp.mean(input, axis=(2, 3))
    logitsT = jnp.einsum("ok,bk->ob", W, gap) + b[:, None]
    h_xT = jax.nn.softmax(logitsT, axis=0)
    score = h_xT[0, :]

    s_row = score.reshape(1, _BATCH)
    s_col = score.reshape(_BATCH, 1)
    rank = pl.pallas_call(
        _rank_body,
        grid=(_BATCH // _RB,),
        in_specs=[
            pl.BlockSpec((1, _BATCH), lambda i: (0, 0)),
            pl.BlockSpec((_RB, 1), lambda i: (i, 0)),
        ],
        out_specs=pl.BlockSpec((_RB, 1), lambda i: (i, 0)),
        out_shape=jax.ShapeDtypeStruct((_BATCH, 1), jnp.int32),
        compiler_params=pltpu.CompilerParams(
            dimension_semantics=("arbitrary",)),
    )(s_row, s_col)

    x3 = (input.reshape(_BATCH, 2, 2, 128, 4, 4)
          .transpose(0, 2, 5, 4, 1, 3)
          .reshape(_BATCH, 8, 1024))
    out3 = pl.pallas_call(
        _apply_body,
        grid=(_BATCH // _BB,),
        in_specs=[
            pl.BlockSpec((_BB, 1), lambda i: (i, 0)),
            pl.BlockSpec((_BB, 8, _D2), lambda i: (i, 0, 0)),
        ],
        out_specs=pl.BlockSpec((_BB, 8, _D2), lambda i: (i, 0, 0)),
        out_shape=jax.ShapeDtypeStruct((_BATCH, 8, 1024), jnp.float32),
        compiler_params=pltpu.CompilerParams(
            dimension_semantics=("parallel",)),
    )(rank, x3)
    return (out3.reshape(_BATCH, 2, 4, 4, 2, 128)
            .transpose(0, 4, 1, 5, 3, 2)
            .reshape(input.shape))


# speedup vs baseline: 3.1344x; 3.1344x over previous
import jax
import jax.numpy as jnp
from jax import lax
from jax.experimental import pallas as pl
from jax.experimental.pallas import tpu as pltpu

_BATCH = 4096
_RB = 256
_BB = 256
_RI = 16

_KS0 = 0
_KS1 = 42
_KS2 = (0x1BD11BDA ^ _KS0 ^ _KS1) & 0xFFFFFFFF
_ROT_A = (13, 15, 26, 6)
_ROT_B = (17, 29, 16, 24)


def _c(v):
    return jnp.int32(v if v < 2**31 else v - 2**32)


def _rotl(x, r):
    return lax.shift_left(x, _c(r)) | lax.shift_right_logical(x, _c(32 - r))


def _mix(x0, x1, rots):
    for r in rots:
        x0 = x0 + x1
        x1 = _rotl(x1, r)
        x1 = x1 ^ x0
    return x0, x1


def _keep_mask(x1):
    x0 = x1
    x1 = _rotl(x1, _ROT_A[0]) ^ x0
    x0, x1 = _mix(x0, x1, _ROT_A[1:])
    x0, x1 = x0 + _c(_KS1), x1 + _c(_KS2 + 1)
    x0, x1 = _mix(x0, x1, _ROT_B)
    x0, x1 = x0 + _c(_KS2), x1 + _c(_KS0 + 2)
    x0, x1 = _mix(x0, x1, _ROT_A)
    x0, x1 = x0 + _c(_KS0), x1 + _c(_KS1 + 3)
    x0, x1 = _mix(x0, x1, _ROT_B)
    x0, x1 = x0 + _c(_KS1), x1 + _c(_KS2 + 4)
    x0, x1 = _mix(x0, x1, _ROT_A)
    x0, x1 = x0 + _c(_KS2), x1 + _c(_KS0 + 5)
    bits = x0 ^ x1
    return bits >= 0


def _rank_body(s_row_ref, s_col_ref, rank_ref):
    i = pl.program_id(0)
    s_all = s_row_ref[...]
    s_col = s_col_ref[...]
    gt = (s_all > s_col).astype(jnp.int32)
    idx_all = lax.broadcasted_iota(jnp.int32, (_RB, _BATCH), 1)
    row_ids = i * _RB + lax.broadcasted_iota(jnp.int32, (_RB, 1), 0)
    tie = ((s_all == s_col) & (idx_all < row_ids)).astype(jnp.int32)
    rank_ref[...] = jnp.sum(gt + tie, axis=1, keepdims=True)


def _apply_body(rank_ref, x_ref, o_ref):
    dh = lax.broadcasted_iota(jnp.int32, (1, 4, 16, 128), 1)
    aew = lax.broadcasted_iota(jnp.int32, (1, 4, 16, 128), 2)
    cm = lax.broadcasted_iota(jnp.int32, (1, 4, 16, 128), 3)
    offs = (lax.shift_left(aew >> 3, _c(12))
            + lax.shift_left((aew >> 2) & 1, _c(11))
            + lax.shift_left(cm, _c(4))
            + lax.shift_left(dh, _c(2))
            + (aew & 3))

    def row(t, carry):
        x1 = jnp.concatenate(
            [(lax.shift_left(rank_ref[_RI * t + u, 0], _c(13)) + _c(_KS1))
             + offs for u in range(_RI)], axis=0)
        keep = _keep_mask(x1)
        x = x_ref[pl.ds(_RI * t, _RI)]
        o_ref[pl.ds(_RI * t, _RI)] = jnp.where(keep, x + x, jnp.zeros_like(x))
        return carry

    lax.fori_loop(0, _BB // _RI, row, 0)


def kernel(input, W, b):
    gap = jnp.mean(input, axis=(2, 3))
    logitsT = jnp.einsum("ok,bk->ob", W, gap) + b[:, None]
    h_xT = jax.nn.softmax(logitsT, axis=0)
    score = h_xT[0, :]

    s_row = score.reshape(1, _BATCH)
    s_col = score.reshape(_BATCH, 1)
    rank = pl.pallas_call(
        _rank_body,
        grid=(_BATCH // _RB,),
        in_specs=[
            pl.BlockSpec((1, _BATCH), lambda i: (0, 0)),
            pl.BlockSpec((_RB, 1), lambda i: (i, 0)),
        ],
        out_specs=pl.BlockSpec((_RB, 1), lambda i: (i, 0)),
        out_shape=jax.ShapeDtypeStruct((_BATCH, 1), jnp.int32),
        compiler_params=pltpu.CompilerParams(
            dimension_semantics=("arbitrary",)),
    )(s_row, s_col)

    x3 = (input.reshape(_BATCH, 2, 2, 128, 4, 4)
          .transpose(0, 4, 1, 2, 5, 3)
          .reshape(_BATCH, 4, 16, 128))
    out3 = pl.pallas_call(
        _apply_body,
        grid=(_BATCH // _BB,),
        in_specs=[
            pl.BlockSpec((_BB, 1), lambda i: (i, 0),
                         memory_space=pltpu.SMEM),
            pl.BlockSpec((_BB, 4, 16, 128), lambda i: (i, 0, 0, 0)),
        ],
        out_specs=pl.BlockSpec((_BB, 4, 16, 128), lambda i: (i, 0, 0, 0)),
        out_shape=jax.ShapeDtypeStruct((_BATCH, 4, 16, 128), jnp.float32),
        compiler_params=pltpu.CompilerParams(
            dimension_semantics=("parallel",)),
    )(rank, x3)
    return (out3.reshape(_BATCH, 4, 2, 2, 4, 128)
            .transpose(0, 2, 3, 5, 1, 4)
            .reshape(input.shape))
